# baseline (speedup 1.0000x reference)
import functools

import jax
import jax.numpy as jnp
from jax import lax
from jax.experimental import pallas as pl
from jax.experimental.pallas import tpu as pltpu
from jax.experimental.pallas import tpu_sc as plsc

N = 10000
E = 320000
HID = 128
OUT = 3
INIT_DIM = 8

NC = 2
NS = 16
NW = NC * NS
EPW = E // NW
NPAD = 10240
RPT = NPAD // NS
DEGW = 16
DCHUNK = 2000


def _make_sc_degree():
    mesh = plsc.VectorSubcoreMesh(core_axis_name="c", subcore_axis_name="s")

    @functools.partial(
        pl.kernel,
        out_type=jax.ShapeDtypeStruct((NC, NPAD, DEGW), jnp.float32),
        mesh=mesh,
        compiler_params=pltpu.CompilerParams(use_tc_tiling_on_sc=False),
        scratch_types=[
            pltpu.VMEM((EPW,), jnp.int32),
            pltpu.VMEM((DCHUNK, DEGW), jnp.float32),
            pltpu.VMEM_SHARED((NPAD, DEGW), jnp.float32),
        ],
    )
    def deg_kernel(ei_hbm, ones_hbm, z_hbm, out_hbm, didx, ones, acc):
        cid = lax.axis_index("c")
        sid = lax.axis_index("s")
        wid = sid * NC + cid
        pltpu.sync_copy(ei_hbm.at[1, pl.ds(wid * EPW, EPW)], didx)
        pltpu.sync_copy(ones_hbm, ones)
        pltpu.sync_copy(z_hbm, acc.at[pl.ds(sid * RPT, RPT)])
        plsc.subcore_barrier()

        def body(j, carry):
            pltpu.sync_copy(
                ones, acc.at[didx.at[pl.ds(j * DCHUNK, DCHUNK)]], add=True)
            return carry

        lax.fori_loop(0, EPW // DCHUNK, body, 0)
        plsc.subcore_barrier()
        pltpu.sync_copy(
            acc.at[pl.ds(sid * RPT, RPT)],
            out_hbm.at[cid, pl.ds(sid * RPT, RPT)],
        )

    return deg_kernel


def _make_sc_propagate(D, CHUNK, col_offs):
    NCHU = EPW // CHUNK
    mesh = plsc.VectorSubcoreMesh(core_axis_name="c", subcore_axis_name="s")

    @functools.partial(
        pl.kernel,
        out_type=jax.ShapeDtypeStruct((NC, NPAD, 128), jnp.float32),
        mesh=mesh,
        compiler_params=pltpu.CompilerParams(use_tc_tiling_on_sc=False),
        scratch_types=[
            pltpu.VMEM((EPW,), jnp.int32),
            pltpu.VMEM((EPW,), jnp.int32),
            pltpu.VMEM((2, CHUNK, D), jnp.float32),
            pltpu.VMEM_SHARED((NPAD, D), jnp.float32),
            pltpu.VMEM_SHARED((NPAD, D), jnp.float32),
            pltpu.SemaphoreType.DMA,
        ],
    )
    def prop_kernel(y_hbm, ei_hbm, z_hbm, out_hbm, sidx, didx, gbuf, tbl, acc,
                    gsem):
        cid = lax.axis_index("c")
        sid = lax.axis_index("s")
        wid = sid * NC + cid
        pltpu.sync_copy(ei_hbm.at[0, pl.ds(wid * EPW, EPW)], sidx)
        pltpu.sync_copy(ei_hbm.at[1, pl.ds(wid * EPW, EPW)], didx)

        def one_pass(c0):
            pltpu.sync_copy(y_hbm.at[pl.ds(sid * RPT, RPT), pl.ds(c0, D)],
                            tbl.at[pl.ds(sid * RPT, RPT)])
            pltpu.sync_copy(z_hbm, acc.at[pl.ds(sid * RPT, RPT)])
            plsc.subcore_barrier()

            pltpu.async_copy(tbl.at[sidx.at[pl.ds(0, CHUNK)]], gbuf.at[0],
                             gsem)

            def chunk(t, carry):
                pltpu.make_async_copy(
                    tbl.at[sidx.at[pl.ds(t * CHUNK, CHUNK)]],
                    gbuf.at[t % 2], gsem).wait()

                @pl.when(t + 1 < NCHU)
                def _prefetch():
                    pltpu.async_copy(
                        tbl.at[sidx.at[pl.ds((t + 1) * CHUNK, CHUNK)]],
                        gbuf.at[(t + 1) % 2], gsem)

                pltpu.sync_copy(
                    gbuf.at[t % 2],
                    acc.at[didx.at[pl.ds(t * CHUNK, CHUNK)]], add=True)
                return carry

            lax.fori_loop(0, NCHU, chunk, 0)
            plsc.subcore_barrier()
            pltpu.sync_copy(
                acc.at[pl.ds(sid * RPT, RPT)],
                out_hbm.at[cid, pl.ds(sid * RPT, RPT), pl.ds(c0, D)],
            )

        for c0 in col_offs:
            one_pass(c0)

    return prop_kernel


_DEG = _make_sc_degree()
_PROP_64 = _make_sc_propagate(64, 200, (0, 64))
_PROP_16 = _make_sc_propagate(16, 1000, (0,))

BM = 2000


def _tc_mm(x, w1):
    def body(xr, wr, xw):
        xw[...] = jnp.dot(xr[...], wr[...], preferred_element_type=jnp.float32)

    return pl.pallas_call(
        body,
        grid=(N // BM,),
        in_specs=[
            pl.BlockSpec((BM, HID), lambda i: (i, 0)),
            pl.BlockSpec((HID, HID), lambda i: (0, 0)),
        ],
        out_specs=pl.BlockSpec((BM, HID), lambda i: (i, 0)),
        out_shape=jax.ShapeDtypeStruct((N, HID), jnp.float32),
    )(x, w1)


def _tc_scale(deg, xw):
    def body(d0, d1, xwr, y_out, dis16):
        dv = d0[0, :, 0:1] + d1[0, :, 0:1] + 1.0
        dis = lax.rsqrt(dv)
        y_out[...] = xwr[...] * dis
        dis16[...] = jnp.broadcast_to(dis, (BM, 16))

    return pl.pallas_call(
        body,
        grid=(N // BM,),
        in_specs=[
            pl.BlockSpec((1, BM, DEGW), lambda i: (0, i, 0)),
            pl.BlockSpec((1, BM, DEGW), lambda i: (1, i, 0)),
            pl.BlockSpec((BM, HID), lambda i: (i, 0)),
        ],
        out_specs=[
            pl.BlockSpec((BM, HID), lambda i: (i, 0)),
            pl.BlockSpec((BM, 16), lambda i: (i, 0)),
        ],
        out_shape=[
            jax.ShapeDtypeStruct((NPAD, HID), jnp.float32),
            jax.ShapeDtypeStruct((N, 16), jnp.float32),
        ],
    )(deg, deg, xw)


def _tc_mid(s, ys, dis16, b1r, w2p):
    def body(a0, a1, yr, dr, br, wr, y2):
        dis = dr[:, 0:1]
        seg = a0[0] + a1[0] + yr[...]
        h = jnp.maximum(dis * seg + br[...], 0.0)
        y2[...] = jnp.dot(h, wr[...], preferred_element_type=jnp.float32) * dis

    slab = lambda cc: pl.BlockSpec((1, BM, HID), lambda i, cc=cc: (cc, i, 0))
    return pl.pallas_call(
        body,
        grid=(N // BM,),
        in_specs=[
            slab(0), slab(1),
            pl.BlockSpec((BM, HID), lambda i: (i, 0)),
            pl.BlockSpec((BM, 16), lambda i: (i, 0)),
            pl.BlockSpec((1, HID), lambda i: (0, 0)),
            pl.BlockSpec((HID, HID), lambda i: (0, 0)),
        ],
        out_specs=pl.BlockSpec((BM, HID), lambda i: (i, 0)),
        out_shape=jax.ShapeDtypeStruct((NPAD, HID), jnp.float32),
    )(s, s, ys, dis16, b1r, w2p)


def _tc_post(t, y2, dis16, init, b2p, wh, wi, bf8):
    def body(a0, a1, yr, dr, ir, br, whr, wir, bfr, out3):
        dis = dr[:, 0:1]
        h2 = dis * (a0[0, :, :16] + a1[0, :, :16] + yr[:, :16]) + br[...]
        res = (
            jnp.dot(h2, whr[...], preferred_element_type=jnp.float32)
            + jnp.dot(ir[...], wir[...], preferred_element_type=jnp.float32)
            + bfr[...]
        )
        out3[...] = res[:, :OUT]

    tslab = lambda cc: pl.BlockSpec((1, BM, 128), lambda i, cc=cc: (cc, i, 0))
    return pl.pallas_call(
        body,
        grid=(N // BM,),
        in_specs=[
            tslab(0), tslab(1),
            pl.BlockSpec((BM, 128), lambda i: (i, 0)),
            pl.BlockSpec((BM, 16), lambda i: (i, 0)),
            pl.BlockSpec((BM, INIT_DIM), lambda i: (i, 0)),
            pl.BlockSpec((1, 16), lambda i: (0, 0)),
            pl.BlockSpec((16, 8), lambda i: (0, 0)),
            pl.BlockSpec((INIT_DIM, 8), lambda i: (0, 0)),
            pl.BlockSpec((1, 8), lambda i: (0, 0)),
        ],
        out_specs=pl.BlockSpec((BM, OUT), lambda i: (i, 0)),
        out_shape=jax.ShapeDtypeStruct((N, OUT), jnp.float32),
    )(t, t, y2, dis16, init, b2p, wh, wi, bf8)


def kernel(x, edge_index, edge_attr, initial_state, W1, b1, W2, b2, Wfc, bfc):
    del edge_attr
    ei = edge_index.astype(jnp.int32)

    b1r = b1.reshape(1, HID)
    w2p = jnp.zeros((HID, HID), jnp.float32).at[:, :OUT].set(W2)
    b2p = jnp.zeros((1, 16), jnp.float32).at[0, :OUT].set(b2)
    wh = jnp.zeros((16, 8), jnp.float32).at[:OUT, :OUT].set(Wfc[:OUT])
    wi = jnp.zeros((INIT_DIM, 8), jnp.float32).at[:, :OUT].set(Wfc[OUT:])
    bf8 = jnp.zeros((1, 8), jnp.float32).at[0, :OUT].set(bfc)

    ones_deg = jnp.ones((DCHUNK, DEGW), jnp.float32)
    z16 = jnp.zeros((RPT, 16), jnp.float32)
    z64 = jnp.zeros((RPT, 64), jnp.float32)

    xw = _tc_mm(x, W1)
    deg = _DEG(ei, ones_deg, z16)
    ys, dis16 = _tc_scale(deg, xw)
    s = _PROP_64(ys, ei, z64)
    y2 = _tc_mid(s, ys, dis16, b1r, w2p)
    t = _PROP_16(y2, ei, z16)
    return _tc_post(t, y2, dis16, initial_state, b2p, wh, wi, bf8)
